# Initial kernel scaffold; baseline (speedup 1.0000x reference)
import jax
import jax.numpy as jnp
from jax.experimental import pallas as pl


def kernel(x, topk_weights, topk_ids, w13_weight, w2_weight):
    raise NotImplementedError("write your pallas kernel here")


# speedup vs baseline: 7.5307x; 7.5307x over previous
import functools

import jax
import jax.numpy as jnp
from jax.experimental import pallas as pl
from jax.experimental.pallas import tpu as pltpu

E = 16
K = 2
D = 1024
F = 512
T = 2048
BT = 512
NT = T // BT


def _moe_dense_kernel(tw_ref, tid_ref, x_ref, w13_ref, w2_ref, out_ref):
    e = pl.program_id(0)
    t = pl.program_id(1)
    xb = x_ref[pl.ds(t * BT, BT), :]
    w13e = w13_ref[0]
    gu = jax.lax.dot_general(xb, w13e, (((1,), (1,)), ((), ())),
                             preferred_element_type=jnp.float32)
    g = gu[:, :F]
    u = gu[:, F:]
    h = g * jax.nn.sigmoid(g) * u
    oe = jax.lax.dot_general(h, w2_ref[0], (((1,), (1,)), ((), ())),
                             preferred_element_type=jnp.float32)
    tid = tid_ref[pl.ds(t * BT, BT), :]
    tw = tw_ref[pl.ds(t * BT, BT), :]
    coeff = jnp.sum(tw * (tid == e).astype(jnp.float32), axis=1, keepdims=True)
    contrib = oe * coeff

    @pl.when(e == 0)
    def _init():
        out_ref[pl.ds(t * BT, BT), :] = contrib

    @pl.when(e != 0)
    def _acc():
        out_ref[pl.ds(t * BT, BT), :] += contrib


@functools.partial(jax.jit, static_argnames=())
def kernel(x, topk_weights, topk_ids, w13_weight, w2_weight):
    grid = (E, NT)
    return pl.pallas_call(
        _moe_dense_kernel,
        grid=grid,
        in_specs=[
            pl.BlockSpec((T, K), lambda e, t: (0, 0)),
            pl.BlockSpec((T, K), lambda e, t: (0, 0)),
            pl.BlockSpec((T, D), lambda e, t: (0, 0)),
            pl.BlockSpec((1, 2 * F, D), lambda e, t: (e, 0, 0)),
            pl.BlockSpec((1, D, F), lambda e, t: (e, 0, 0)),
        ],
        out_specs=pl.BlockSpec((T, D), lambda e, t: (0, 0)),
        out_shape=jax.ShapeDtypeStruct((T, D), jnp.float32),
        compiler_params=pltpu.CompilerParams(
            dimension_semantics=("arbitrary", "arbitrary"),
        ),
    )(topk_weights, topk_ids, x, w13_weight, w2_weight)
